# baseline (speedup 1.0000x reference)
import jax
import jax.numpy as jnp
from jax import lax
from jax.experimental import pallas as pl

G = 16
NEG_SLOPE = 0.2


def _tc_layer0(x2d, embed, W, asrc2d, adst2d):
    N = x2d.shape[0]
    H = W.shape[0]

    def body(x_ref, e_ref, w_ref, as_ref, ad_ref, hp_ref, als_ref, ald_ref):
        xb = x_ref[...]
        h = jnp.where(xb == 0, e_ref[0:1, :], e_ref[1:2, :])
        hp = jnp.dot(h, w_ref[...], preferred_element_type=jnp.float32)
        hp_ref[...] = hp
        als_ref[...] = jnp.sum(hp * as_ref[...], axis=1, keepdims=True)
        ald_ref[...] = jnp.sum(hp * ad_ref[...], axis=1, keepdims=True)

    return pl.pallas_call(
        body,
        out_shape=[
            jax.ShapeDtypeStruct((N, H), jnp.float32),
            jax.ShapeDtypeStruct((N, 1), jnp.float32),
            jax.ShapeDtypeStruct((N, 1), jnp.float32),
        ],
    )(x2d, embed, W, asrc2d, adst2d)


def _tc_layer(num, den, b2d, W, asrc2d, adst2d, relu):
    N = num.shape[0]
    H = W.shape[0]

    def body(n_ref, d_ref, b_ref, w_ref, as_ref, ad_ref,
             hp_ref, als_ref, ald_ref):
        h = n_ref[...] / (d_ref[...] + 1e-16) + b_ref[...]
        if relu:
            h = jnp.maximum(h, 0.0)
        hp = jnp.dot(h, w_ref[...], preferred_element_type=jnp.float32)
        hp_ref[...] = hp
        als_ref[...] = jnp.sum(hp * as_ref[...], axis=1, keepdims=True)
        ald_ref[...] = jnp.sum(hp * ad_ref[...], axis=1, keepdims=True)

    return pl.pallas_call(
        body,
        out_shape=[
            jax.ShapeDtypeStruct((N, H), jnp.float32),
            jax.ShapeDtypeStruct((N, 1), jnp.float32),
            jax.ShapeDtypeStruct((N, 1), jnp.float32),
        ],
    )(num, den, b2d, W, asrc2d, adst2d)


def _tc_final(num, den, b2d, batch2d, W_out, bout2d):
    C = W_out.shape[1]

    def body(n_ref, d_ref, b_ref, bt_ref, wo_ref, bo_ref, o_ref):
        h = n_ref[...] / (d_ref[...] + 1e-16) + b_ref[...]
        gids = lax.broadcasted_iota(jnp.int32, (G, 1), 0)
        P = (gids == bt_ref[...]).astype(jnp.float32)
        sums = jnp.dot(P, h, preferred_element_type=jnp.float32)
        cnt = jnp.sum(P, axis=1, keepdims=True)
        pooled = sums / jnp.maximum(cnt, 1.0)
        o_ref[...] = (jnp.dot(pooled, wo_ref[...],
                              preferred_element_type=jnp.float32)
                      + bo_ref[...])

    return pl.pallas_call(
        body,
        out_shape=jax.ShapeDtypeStruct((G, C), jnp.float32),
    )(num, den, b2d, batch2d, W_out, bout2d)


def _edge_phase(hp, als, ald, src, dst, N):
    e = als[src, 0] + ald[dst, 0]
    ex = jnp.where(e >= 0.0, jnp.exp(e), jnp.exp(e * NEG_SLOPE))
    num = jax.ops.segment_sum(hp[src] * ex[:, None], dst, num_segments=N)
    den = jax.ops.segment_sum(ex, dst, num_segments=N)[:, None]
    return num, den


def kernel(x, edge_index, batch, embed, Ws, a_src, a_dst, bs, W_out, b_out):
    N = x.shape[0]
    L, H, _ = Ws.shape
    C = W_out.shape[1]
    src = edge_index[0]
    dst = edge_index[1]

    x2d = x.astype(jnp.int32)
    hp, als, ald = _tc_layer0(x2d, embed, Ws[0],
                              a_src[0].reshape(1, H), a_dst[0].reshape(1, H))
    num, den = _edge_phase(hp, als, ald, src, dst, N)

    for i in range(1, L):
        hp, als, ald = _tc_layer(num, den, bs[i - 1].reshape(1, H), Ws[i],
                                 a_src[i].reshape(1, H),
                                 a_dst[i].reshape(1, H), relu=True)
        num, den = _edge_phase(hp, als, ald, src, dst, N)

    return _tc_final(num, den, bs[L - 1].reshape(1, H),
                     batch.reshape(1, N).astype(jnp.int32),
                     W_out, b_out.reshape(1, C))
